# baseline (speedup 1.0000x reference)
import functools

import jax
import jax.numpy as jnp
from jax import lax
from jax.experimental import pallas as pl
from jax.experimental.pallas import tpu as pltpu
from jax.experimental.pallas import tpu_sc as plsc

NC = 2
NS = 16
NW = NC * NS
LANES = 16


def _make_sc_kernel(B, L, V, D):
    LT = L // 8
    BT = B // 128
    DT = D // 8
    bt_per_w = BT // NW
    n_chunks = LT * bt_per_w * 2
    mesh = plsc.VectorSubcoreMesh(core_axis_name="c", subcore_axis_name="s")

    @functools.partial(
        pl.kernel,
        out_type=jax.ShapeDtypeStruct((L, DT, BT, 8, 128), jnp.float32),
        mesh=mesh,
        compiler_params=pltpu.CompilerParams(use_tc_tiling_on_sc=False,
                                             needs_layout_passes=False),
        scratch_types=(
            [pltpu.VMEM((4, 128), jnp.int32) for _ in range(4)] +
            [pltpu.VMEM((4, 128), jnp.int32) for _ in range(4)] +
            [pltpu.VMEM((512, D), jnp.float32) for _ in range(4)] +
            [pltpu.VMEM((128, 129), jnp.float32) for _ in range(2)] +
            [pltpu.VMEM((2, D), jnp.float32)] +
            [pltpu.SemaphoreType.DMA for _ in range(4)] +
            [pltpu.SemaphoreType.DMA for _ in range(4)] +
            [pltpu.SemaphoreType.DMA for _ in range(4)] +
            [pltpu.SemaphoreType.DMA for _ in range(2)]
        ),
    )
    def k(vt_hbm, vv_hbm, bool_hbm, pred_hbm, out_hbm, *scr):
        idx = list(scr[0:4])
        val = list(scr[4:8])
        rr = list(scr[8:12])
        tt = list(scr[12:14])
        bool_v = scr[14]
        sg = list(scr[15:19])
        si = list(scr[19:23])
        sv = list(scr[23:27])
        so = list(scr[27:29])

        wid = lax.axis_index("s") * NC + lax.axis_index("c")
        bt0 = wid * bt_per_w
        pltpu.sync_copy(bool_hbm, bool_v)
        iota16 = lax.iota(jnp.int32, LANES)
        rowpat = [(2 * h + iota16 // 8) * 8 + iota16 % 8 for h in range(2)]
        b0 = [bool_v[0, pl.ds(0, LANES)], bool_v[0, pl.ds(LANES, LANES)]]
        b1 = [bool_v[1, pl.ds(0, LANES)], bool_v[1, pl.ds(LANES, LANES)]]

        def coords(g):
            lt = g // (2 * bt_per_w)
            rem = g % (2 * bt_per_w)
            bt = bt0 + rem // 2
            h = rem % 2
            return lt, bt, h

        def fire_idx(g, b):
            lt, bt, h = coords(g)
            pltpu.async_copy(vt_hbm.at[lt, bt, pl.ds(4 * h, 4)], idx[b], si[b])

        def fire_val(g, b):
            lt, bt, h = coords(g)
            pltpu.async_copy(vv_hbm.at[lt, bt, pl.ds(4 * h, 4)], val[b], sv[b])

        def wait_idx(b):
            pltpu.make_async_copy(
                vt_hbm.at[0, 0, pl.ds(0, 4)], idx[b], si[b]).wait()

        def wait_val(b):
            pltpu.make_async_copy(
                vv_hbm.at[0, 0, pl.ds(0, 4)], val[b], sv[b]).wait()

        def fire_gather(b):
            for lr in range(4):
                pltpu.async_copy(pred_hbm.at[idx[b].at[lr]],
                                 rr[b].at[pl.ds(128 * lr, 128)], sg[b])

        def wait_gather(b):
            for lr in range(4):
                pltpu.make_async_copy(pred_hbm.at[idx[b].at[lr]],
                                      rr[b].at[pl.ds(128 * lr, 128)],
                                      sg[b]).wait()

        def step(g, bi, bo):
            wait_gather(bi)

            @pl.when(g + 4 < n_chunks)
            def _():
                fire_idx(g + 4, bi)

            @pl.when(g + 2 < n_chunks)
            def _():
                wait_idx((bi + 2) % 4)
                fire_gather((bi + 2) % 4)

            @pl.when(g >= 2)
            def _():
                for p in range(16):
                    pltpu.make_async_copy(
                        tt[bo].at[pl.ds(8 * p, 8), pl.ds(0, 128)],
                        out_hbm.at[0, 0, 0], so[bo]).wait()

            wait_val(bi)

            @plsc.parallel_loop(0, 32, unroll=2)
            def q_body(q):
                lr = q // 8
                j = q % 8
                valv = val[bi][lr, pl.ds(LANES * j, LANES)]
                rowv = [rowpat[0] + lr * 32, rowpat[1] + lr * 32]
                for jj in range(LANES):
                    vb = (jnp.full((LANES,), 0, jnp.int32) + valv[jj]) != 0
                    colv = jnp.full((LANES,), 0, jnp.int32) + (LANES * j + jj)
                    for h in range(2):
                        rv = rr[bi][LANES * q + jj, pl.ds(LANES * h, LANES)]
                        ov = rv * jnp.where(vb, b1[h], b0[h])
                        plsc.store_scatter(tt[bo], [rowv[h], colv], ov)

            @pl.when(g + 4 < n_chunks)
            def _():
                fire_val(g + 4, bi)

            lt, bt, h = coords(g)
            l0 = 8 * lt + 4 * h
            for lr in range(4):
                for t in range(DT):
                    pltpu.async_copy(
                        tt[bo].at[pl.ds(32 * lr + 8 * t, 8), pl.ds(0, 128)],
                        out_hbm.at[l0 + lr, t, bt], so[bo])

        for p in range(4):
            fire_idx(p, p)
            fire_val(p, p)
        wait_idx(0)
        fire_gather(0)
        wait_idx(1)
        fire_gather(1)

        def quad_body(cc, _):
            for kk in range(4):
                step(4 * cc + kk, kk, kk % 2)
            return ()

        lax.fori_loop(0, n_chunks // 4, quad_body, ())

        for b in range(2):
            for p in range(16):
                pltpu.make_async_copy(
                    tt[b].at[pl.ds(8 * p, 8), pl.ds(0, 128)],
                    out_hbm.at[0, 0, 0], so[b]).wait()

    return k


def kernel(var_val, var_type, boolean_table, pred_table):
    B, L = var_val.shape
    V, D = pred_table.shape
    vt = var_type.reshape(B // 128, 128, L // 8, 8).transpose(2, 0, 3, 1)
    vv = var_val.reshape(B // 128, 128, L // 8, 8).transpose(2, 0, 3, 1)
    k = _make_sc_kernel(B, L, V, D)
    y = k(vt, vv, boolean_table, pred_table)
    return y.transpose(2, 4, 0, 1, 3).reshape(B, L, D)


# speedup vs baseline: 1.4305x; 1.4305x over previous
import functools

import jax
import jax.numpy as jnp
from jax import lax
from jax.experimental import pallas as pl
from jax.experimental.pallas import tpu as pltpu
from jax.experimental.pallas import tpu_sc as plsc

NC = 2
NS = 16
NW = NC * NS
LANES = 16


def _make_sc_kernel(B, L, V, D):
    LT = L // 8
    BT = B // 128
    DT = D // 8
    bt_per_w = BT // NW
    n_chunks = LT * bt_per_w * 2
    mesh = plsc.VectorSubcoreMesh(core_axis_name="c", subcore_axis_name="s")

    @functools.partial(
        pl.kernel,
        out_type=jax.ShapeDtypeStruct((L, DT, BT, 8, 128), jnp.float32),
        mesh=mesh,
        compiler_params=pltpu.CompilerParams(use_tc_tiling_on_sc=False,
                                             needs_layout_passes=False),
        scratch_types=(
            [pltpu.VMEM((4, 128), jnp.int32) for _ in range(4)] +
            [pltpu.VMEM((4, 128), jnp.int32) for _ in range(4)] +
            [pltpu.VMEM((512, D), jnp.float32) for _ in range(4)] +
            [pltpu.VMEM((128, 129), jnp.float32) for _ in range(2)] +
            [pltpu.VMEM((2, D), jnp.float32)] +
            [pltpu.SemaphoreType.DMA for _ in range(4)] +
            [pltpu.SemaphoreType.DMA for _ in range(4)] +
            [pltpu.SemaphoreType.DMA for _ in range(4)] +
            [pltpu.SemaphoreType.DMA for _ in range(2)]
        ),
    )
    def k(vt_hbm, vv_hbm, bool_hbm, pred_hbm, out_hbm, *scr):
        idx = list(scr[0:4])
        val = list(scr[4:8])
        rr = list(scr[8:12])
        tt = list(scr[12:14])
        bool_v = scr[14]
        sg = list(scr[15:19])
        si = list(scr[19:23])
        sv = list(scr[23:27])
        so = list(scr[27:29])

        wid = lax.axis_index("s") * NC + lax.axis_index("c")
        bt0 = wid * bt_per_w
        pltpu.sync_copy(bool_hbm, bool_v)
        iota16 = lax.iota(jnp.int32, LANES)
        rowpat = [(2 * h + iota16 // 8) * 8 + iota16 % 8 for h in range(2)]
        b0 = [bool_v[0, pl.ds(0, LANES)], bool_v[0, pl.ds(LANES, LANES)]]
        bd = [bool_v[1, pl.ds(0, LANES)] - b0[0],
              bool_v[1, pl.ds(LANES, LANES)] - b0[1]]

        def coords(g):
            lt = g // (2 * bt_per_w)
            rem = g % (2 * bt_per_w)
            bt = bt0 + rem // 2
            h = rem % 2
            return lt, bt, h

        def fire_idx(g, b):
            lt, bt, h = coords(g)
            pltpu.async_copy(vt_hbm.at[lt, bt, pl.ds(4 * h, 4)], idx[b], si[b])

        def fire_val(g, b):
            lt, bt, h = coords(g)
            pltpu.async_copy(vv_hbm.at[lt, bt, pl.ds(4 * h, 4)], val[b], sv[b])

        def wait_idx(b):
            pltpu.make_async_copy(
                vt_hbm.at[0, 0, pl.ds(0, 4)], idx[b], si[b]).wait()

        def wait_val(b):
            pltpu.make_async_copy(
                vv_hbm.at[0, 0, pl.ds(0, 4)], val[b], sv[b]).wait()

        def fire_gather(b):
            for lr in range(4):
                pltpu.async_copy(pred_hbm.at[idx[b].at[lr]],
                                 rr[b].at[pl.ds(128 * lr, 128)], sg[b])

        def wait_gather(b):
            for lr in range(4):
                pltpu.make_async_copy(pred_hbm.at[idx[b].at[lr]],
                                      rr[b].at[pl.ds(128 * lr, 128)],
                                      sg[b]).wait()

        def step(g, bi, bo):
            wait_gather(bi)

            @pl.when(g + 4 < n_chunks)
            def _():
                fire_idx(g + 4, bi)

            @pl.when(g + 2 < n_chunks)
            def _():
                wait_idx((bi + 2) % 4)
                fire_gather((bi + 2) % 4)

            @pl.when(g >= 2)
            def _():
                for p in range(16):
                    pltpu.make_async_copy(
                        tt[bo].at[pl.ds(8 * p, 8), pl.ds(0, 128)],
                        out_hbm.at[0, 0, 0], so[bo]).wait()

            wait_val(bi)

            @plsc.parallel_loop(0, 32, unroll=2)
            def q_body(q):
                lr = q // 8
                j = q % 8
                valv = val[bi][lr, pl.ds(LANES * j, LANES)].astype(
                    jnp.float32)
                rowv = [rowpat[0] + lr * 32, rowpat[1] + lr * 32]
                for jj in range(LANES):
                    vf = valv[jj]
                    colv = jnp.full((LANES,), 0, jnp.int32) + (LANES * j + jj)
                    for h in range(2):
                        rv = rr[bi][LANES * q + jj, pl.ds(LANES * h, LANES)]
                        ov = rv * (b0[h] + vf * bd[h])
                        plsc.store_scatter(tt[bo], [rowv[h], colv], ov)

            @pl.when(g + 4 < n_chunks)
            def _():
                fire_val(g + 4, bi)

            lt, bt, h = coords(g)
            l0 = 8 * lt + 4 * h
            for lr in range(4):
                for t in range(DT):
                    pltpu.async_copy(
                        tt[bo].at[pl.ds(32 * lr + 8 * t, 8), pl.ds(0, 128)],
                        out_hbm.at[l0 + lr, t, bt], so[bo])

        for p in range(4):
            fire_idx(p, p)
            fire_val(p, p)
        wait_idx(0)
        fire_gather(0)
        wait_idx(1)
        fire_gather(1)

        def quad_body(cc, _):
            for kk in range(4):
                step(4 * cc + kk, kk, kk % 2)
            return ()

        lax.fori_loop(0, n_chunks // 4, quad_body, ())

        for b in range(2):
            for p in range(16):
                pltpu.make_async_copy(
                    tt[b].at[pl.ds(8 * p, 8), pl.ds(0, 128)],
                    out_hbm.at[0, 0, 0], so[b]).wait()

    return k


def kernel(var_val, var_type, boolean_table, pred_table):
    B, L = var_val.shape
    V, D = pred_table.shape
    vt = var_type.reshape(B // 128, 128, L // 8, 8).transpose(2, 0, 3, 1)
    vv = var_val.reshape(B // 128, 128, L // 8, 8).transpose(2, 0, 3, 1)
    k = _make_sc_kernel(B, L, V, D)
    y = k(vt, vv, boolean_table, pred_table)
    return y.transpose(2, 4, 0, 1, 3).reshape(B, L, D)
